# baseline (speedup 1.0000x reference)
import functools
import math

import jax
import jax.numpy as jnp
from jax import lax
from jax.experimental import pallas as pl
from jax.experimental.pallas import tpu as pltpu
from jax.experimental.pallas import tpu_sc as plsc

_DIM = 64
_SCALE = math.sqrt(_DIM)
_LANES = 16

_NC = 2
_NS = 16
_NW = _NC * _NS

_IDXW = 128
_CHUNK = 1024
_GPC = _CHUNK // _IDXW


def _make_lookup(n_idx):
    assert n_idx % (_NW * _CHUNK) == 0
    per_w = n_idx // _NW
    n_chunks = per_w // _CHUNK
    mesh = plsc.VectorSubcoreMesh(core_axis_name="c", subcore_axis_name="s")

    @functools.partial(
        pl.kernel,
        mesh=mesh,
        out_type=jax.ShapeDtypeStruct((n_idx, _DIM), jnp.float32),
        scratch_types=[
            pltpu.VMEM((_GPC, _IDXW), jnp.int32),
            pltpu.VMEM((_CHUNK, _DIM), jnp.float32),
            pltpu.SemaphoreType.DMA,
        ],
        compiler_params=pltpu.CompilerParams(use_tc_tiling_on_sc=False),
    )
    def lookup(idx_hbm, table_hbm, out_hbm, idx_v, rows_v, sem):
        wid = lax.axis_index("s") * _NC + lax.axis_index("c")
        wbase = wid * per_w

        def chunk_body(g, carry):
            base = wbase + g * _CHUNK
            row0 = pl.multiple_of(base // _IDXW, 8)
            pltpu.sync_copy(idx_hbm.at[pl.ds(row0, _GPC)], idx_v)
            copies = [
                pltpu.async_copy(
                    table_hbm.at[idx_v.at[j]],
                    rows_v.at[pl.ds(j * _IDXW, _IDXW)],
                    sem,
                )
                for j in range(_GPC)
            ]
            for c in copies:
                c.wait()

            def scale_row(r, c2):
                for cc in range(_DIM // _LANES):
                    sl = pl.ds(cc * _LANES, _LANES)
                    rows_v[r, sl] = rows_v[r, sl] * _SCALE
                return c2

            lax.fori_loop(0, _CHUNK, scale_row, 0)
            pltpu.sync_copy(rows_v, out_hbm.at[pl.ds(base, _CHUNK)])
            return carry

        lax.fori_loop(0, n_chunks, chunk_body, 0)

    return lookup


def kernel(x, table):
    b0, b1 = x.shape
    n_idx = b0 * b1
    idx2d = x.reshape(n_idx // _IDXW, _IDXW).astype(jnp.int32)
    out = _make_lookup(n_idx)(idx2d, table)
    return out.reshape(b0, b1, _DIM)


# speedup vs baseline: 1.1107x; 1.1107x over previous
import functools
import math

import jax
import jax.numpy as jnp
from jax import lax
from jax.experimental import pallas as pl
from jax.experimental.pallas import tpu as pltpu
from jax.experimental.pallas import tpu_sc as plsc

_DIM = 64
_SCALE = math.sqrt(_DIM)
_LANES = 16

_NC = 2
_NS = 16
_NW = _NC * _NS

_IDXW = 128
_CHUNK = 256
_GPC = _CHUNK // _IDXW
_NBUF = 4


def _make_lookup(n_idx):
    assert n_idx % (_NW * _CHUNK) == 0
    per_w = n_idx // _NW
    idx_rows = per_w // _IDXW
    n_chunks = per_w // _CHUNK
    assert idx_rows % 8 == 0 and n_chunks >= _NBUF
    mesh = plsc.VectorSubcoreMesh(core_axis_name="c", subcore_axis_name="s")

    @functools.partial(
        pl.kernel,
        mesh=mesh,
        out_type=jax.ShapeDtypeStruct((n_idx, _DIM), jnp.float32),
        scratch_types=[
            pltpu.VMEM((idx_rows, _IDXW), jnp.int32),
            pltpu.VMEM((_NBUF, _CHUNK, _DIM), jnp.float32),
            pltpu.SemaphoreType.DMA((_NBUF,)),
            pltpu.SemaphoreType.DMA((_NBUF,)),
        ],
        compiler_params=pltpu.CompilerParams(use_tc_tiling_on_sc=False),
    )
    def lookup(idx_hbm, table_hbm, out_hbm, idx_v, rows_v, gsem, osem):
        wid = lax.axis_index("s") * _NC + lax.axis_index("c")
        wbase = wid * per_w
        wrow0 = pl.multiple_of(wid * idx_rows, 8)

        pltpu.sync_copy(idx_hbm.at[pl.ds(wrow0, idx_rows)], idx_v)

        def gather_chunk(t, start):
            s = lax.rem(t, _NBUF)
            copies = []
            for j in range(_GPC):
                c = pltpu.make_async_copy(
                    table_hbm.at[idx_v.at[t * _GPC + j]],
                    rows_v.at[s, pl.ds(j * _IDXW, _IDXW)],
                    gsem.at[s],
                )
                if start:
                    c.start()
                copies.append(c)
            return copies

        def out_copy(t):
            s = lax.rem(t, _NBUF)
            return pltpu.make_async_copy(
                rows_v.at[s],
                out_hbm.at[pl.ds(wbase + t * _CHUNK, _CHUNK)],
                osem.at[s],
            )

        gather_chunk(0, True)
        gather_chunk(1, True)

        def chunk_body(t, carry):
            s = lax.rem(t, _NBUF)

            @pl.when(t + 2 < n_chunks)
            def _fire_ahead():
                @pl.when(t >= 2)
                def _drain_out():
                    out_copy(t - 2).wait()

                gather_chunk(t + 2, True)

            for c in gather_chunk(t, False):
                c.wait()

            @plsc.parallel_loop(0, _CHUNK, step=1, unroll=8)
            def _scale(r):
                for cc in range(_DIM // _LANES):
                    sl = pl.ds(cc * _LANES, _LANES)
                    rows_v[s, r, sl] = rows_v[s, r, sl] * _SCALE

            out_copy(t).start()
            return carry

        lax.fori_loop(0, n_chunks, chunk_body, 0)

        for t in range(n_chunks - _NBUF, n_chunks):
            out_copy(t).wait()

    return lookup


def kernel(x, table):
    b0, b1 = x.shape
    n_idx = b0 * b1
    idx2d = x.reshape(n_idx // _IDXW, _IDXW).astype(jnp.int32)
    out = _make_lookup(n_idx)(idx2d, table)
    return out.reshape(b0, b1, _DIM)


# speedup vs baseline: 1.1385x; 1.0250x over previous
import functools
import math

import jax
import jax.numpy as jnp
from jax import lax
from jax.experimental import pallas as pl
from jax.experimental.pallas import tpu as pltpu
from jax.experimental.pallas import tpu_sc as plsc

_DIM = 64
_PADDIM = 128
_SCALE = math.sqrt(_DIM)
_LANES = 16

_NC = 2
_NS = 16
_NW = _NC * _NS

_GBUF = 4
_OBUF = 2


def _make_lookup(n_sent, seq):
    assert n_sent % _NW == 0
    sent_w = n_sent // _NW
    idx_w = sent_w * seq
    mesh = plsc.VectorSubcoreMesh(core_axis_name="c", subcore_axis_name="s")
    vecs = (seq + _LANES - 1) // _LANES

    @functools.partial(
        pl.kernel,
        mesh=mesh,
        out_type=jax.ShapeDtypeStruct((n_sent, seq, _DIM), jnp.float32),
        scratch_types=[
            pltpu.VMEM((idx_w + _LANES,), jnp.int32),
            pltpu.VMEM((_GBUF, vecs * _LANES, _PADDIM), jnp.float32),
            pltpu.VMEM((_OBUF, seq, _DIM), jnp.float32),
            pltpu.SemaphoreType.DMA((_GBUF,)),
            pltpu.SemaphoreType.DMA((_OBUF,)),
        ],
        compiler_params=pltpu.CompilerParams(use_tc_tiling_on_sc=True),
    )
    def lookup(idx_hbm, table_hbm, out_hbm, idx_v, rows_v, stage_v, gsem, osem):
        wid = lax.axis_index("s") * _NC + lax.axis_index("c")
        wbase = wid * idx_w

        pltpu.sync_copy(idx_hbm.at[pl.ds(wbase, idx_w)], idx_v.at[pl.ds(0, idx_w)])
        idx_v[pl.ds(idx_w, _LANES)] = jnp.zeros((_LANES,), jnp.int32)

        iota = lax.iota(jnp.int32, _LANES)

        def gather_sent(t, start):
            s = lax.rem(t, _GBUF)
            copies = []
            for j in range(vecs):
                iv = idx_v[pl.ds(t * seq + j * _LANES, _LANES)]
                c = pltpu.make_async_copy(
                    table_hbm.at[iv],
                    rows_v.at[s, pl.ds(j * _LANES, _LANES)],
                    gsem.at[s],
                )
                if start:
                    c.start()
                copies.append(c)
            return copies

        def out_copy(t):
            ss = lax.rem(t, _OBUF)
            return pltpu.make_async_copy(
                stage_v.at[ss],
                out_hbm.at[wid * sent_w + t],
                osem.at[ss],
            )

        gather_sent(0, True)
        gather_sent(1, True)

        def sent_body(t, carry):
            s = lax.rem(t, _GBUF)
            ss = lax.rem(t, _OBUF)

            @pl.when(t + 2 < sent_w)
            def _fire_ahead():
                gather_sent(t + 2, True)

            for c in gather_sent(t, False):
                c.wait()

            @pl.when(t >= _OBUF)
            def _drain_out():
                out_copy(t - _OBUF).wait()

            @plsc.parallel_loop(0, seq, step=1, unroll=8)
            def _scale(r):
                for cc in range(_DIM // _LANES):
                    sl = pl.ds(cc * _LANES, _LANES)
                    stage_v[ss, r, sl] = rows_v[s, r, sl] * _SCALE

            out_copy(t).start()
            return carry

        lax.fori_loop(0, sent_w, sent_body, 0)

        for t in range(sent_w - _OBUF, sent_w):
            out_copy(t).wait()

    return lookup


def kernel(x, table):
    n_sent, seq = x.shape
    idx_flat = x.reshape(n_sent * seq).astype(jnp.int32)
    table_pad = jnp.pad(table, ((0, 0), (0, _PADDIM - _DIM)))
    return _make_lookup(n_sent, seq)(idx_flat, table_pad)
